# baseline (speedup 1.0000x reference)
import functools

import jax
import jax.numpy as jnp
from jax import lax
from jax.experimental import pallas as pl
from jax.experimental.pallas import tpu as pltpu
from jax.experimental.pallas import tpu_sc as plsc

N_NODES = 10000
E = 320000
D = 128
H = 128
B = 1024

NC = 2
NS = 16
NPAD = 10240
EP = E // NS
EH = EP // 2
SL = NPAD // NS
L = 16

_mesh = plsc.VectorSubcoreMesh(
    core_axis_name="c", subcore_axis_name="s", num_cores=NC, num_subcores=NS
)


@functools.partial(
    pl.kernel,
    out_type=jax.ShapeDtypeStruct((3 * NPAD,), jnp.float32),
    mesh=_mesh,
    compiler_params=pltpu.CompilerParams(needs_layout_passes=False),
    scratch_types=[
        pltpu.VMEM((EP,), jnp.int32),
        pltpu.VMEM((EP,), jnp.int32),
        pltpu.VMEM((NPAD,), jnp.int32),
        pltpu.VMEM((NPAD,), jnp.int32),
        pltpu.VMEM((NPAD,), jnp.int32),
        pltpu.VMEM((B,), jnp.int32),
        pltpu.VMEM((NS, SL), jnp.int32),
        pltpu.VMEM((SL,), jnp.int32),
        pltpu.VMEM((SL,), jnp.int32),
        pltpu.VMEM((SL,), jnp.float32),
        pltpu.VMEM((SL,), jnp.float32),
        pltpu.VMEM_SHARED((NS, NPAD), jnp.int32),
        pltpu.VMEM_SHARED((NPAD,), jnp.int32),
        pltpu.SemaphoreType.DMA,
        pltpu.SemaphoreType.DMA,
        pltpu.SemaphoreType.DMA,
    ],
)
def _sc_bfs_masks(edges_hbm, ids_hbm, out_hbm,
                  src_v, dst_v, mask0_v, f1_v, f2_v, ids_v,
                  slice_v, comb1_v, comb2_v, oa_v, ob_v,
                  stage_sh, comb_sh, sem_a, sem_b, sem_i):
    cid = lax.axis_index("c")
    tid = lax.axis_index("s")
    ebase = pl.multiple_of(tid * EP, 8)
    nbase = pl.multiple_of(tid * SL, 8)

    cp_s1 = pltpu.async_copy(edges_hbm.at[pl.ds(ebase, EH)],
                             src_v.at[pl.ds(0, EH)], sem_a)
    cp_d1 = pltpu.async_copy(edges_hbm.at[pl.ds(E + ebase, EH)],
                             dst_v.at[pl.ds(0, EH)], sem_a)
    cp_s2 = pltpu.async_copy(edges_hbm.at[pl.ds(ebase + EH, EH)],
                             src_v.at[pl.ds(EH, EH)], sem_b)
    cp_d2 = pltpu.async_copy(edges_hbm.at[pl.ds(E + ebase + EH, EH)],
                             dst_v.at[pl.ds(EH, EH)], sem_b)
    cp_i = pltpu.async_copy(ids_hbm, ids_v, sem_i)

    zeros = jnp.zeros((L,), jnp.int32)
    ones = jnp.ones((L,), jnp.int32)

    @plsc.parallel_loop(0, NPAD, step=L, unroll=8)
    def _(i):
        off = pl.multiple_of(i, 8)
        mask0_v[pl.ds(off, L)] = zeros
        f1_v[pl.ds(off, L)] = zeros
        f2_v[pl.ds(off, L)] = zeros

    cp_i.wait()

    @plsc.parallel_loop(0, B, step=L, unroll=4)
    def _(i):
        off = pl.multiple_of(i, 8)
        plsc.store_scatter(mask0_v, [ids_v[pl.ds(off, L)]], ones)

    def edge_pass(cur_ref, front_ref, lo, hi):
        @plsc.parallel_loop(lo, hi, step=L, unroll=16)
        def _(i):
            off = pl.multiple_of(i, 8)
            sv = src_v[pl.ds(off, L)]
            dv = dst_v[pl.ds(off, L)]
            fr = plsc.load_gather(cur_ref, [sv])
            plsc.store_scatter(front_ref, [dv], ones, mask=fr > 0)

    def combine(front_ref, comb_slice_ref):
        pltpu.sync_copy(front_ref, stage_sh.at[tid])
        plsc.subcore_barrier()
        pltpu.sync_copy(stage_sh.at[:, pl.ds(nbase, SL)], slice_v)

        @plsc.parallel_loop(0, SL, step=L, unroll=4)
        def _(j):
            joff = pl.multiple_of(j, 8)
            acc = slice_v[0, pl.ds(joff, L)]
            for r in range(1, NS):
                acc = jnp.maximum(acc, slice_v[r, pl.ds(joff, L)])
            comb_slice_ref[pl.ds(joff, L)] = acc

    cp_s1.wait()
    cp_d1.wait()
    edge_pass(mask0_v, f1_v, 0, EH)
    cp_s2.wait()
    cp_d2.wait()
    edge_pass(mask0_v, f1_v, EH, EP)
    combine(f1_v, comb1_v)
    pltpu.sync_copy(comb1_v, comb_sh.at[pl.ds(nbase, SL)])
    plsc.subcore_barrier()
    pltpu.sync_copy(comb_sh, f1_v)

    half = pl.multiple_of(cid * EH, 8)
    edge_pass(f1_v, f2_v, half, half + EH)
    combine(f2_v, comb2_v)

    @plsc.parallel_loop(0, SL, step=L, unroll=4)
    def _(j):
        joff = pl.multiple_of(j, 8)
        a = comb1_v[pl.ds(joff, L)]
        u = jnp.maximum(a, comb2_v[pl.ds(joff, L)])
        oa_v[pl.ds(joff, L)] = a.astype(jnp.float32)
        ob_v[pl.ds(joff, L)] = u.astype(jnp.float32)

    @pl.when(cid == 0)
    def _():
        pltpu.sync_copy(oa_v, out_hbm.at[pl.ds(nbase, SL)])

    ubase = pl.multiple_of((1 + cid) * NPAD + tid * SL, 8)
    pltpu.sync_copy(ob_v, out_hbm.at[pl.ds(ubase, SL)])


def _tc_body(masks_ref, emb_ref, w0_ref, b0_ref, w1_ref, b1_ref,
             wo_ref, bo_ref, out_ref):
    hi = jax.lax.Precision.HIGHEST
    m3 = masks_ref[...]
    m0 = m3[0:1]
    m1 = jnp.maximum(m3[0:1], jnp.maximum(m3[1:2], m3[2:3]))
    masks = jnp.concatenate([m0, m1], axis=0)
    cnt = jnp.sum(masks, axis=1)
    sums = lax.dot_general(masks[:, :N_NODES], emb_ref[...],
                           (((1,), (0,)), ((), ())), precision=hi)
    agg = jnp.where((cnt > 0)[:, None],
                    sums / jnp.maximum(cnt, 1.0)[:, None], 0.0)
    h0 = jnp.maximum(
        lax.dot_general(agg[0:1], w0_ref[...], (((1,), (1,)), ((), ())),
                        precision=hi) + b0_ref[...], 0.0)
    h1 = jnp.maximum(
        lax.dot_general(agg[1:2], w1_ref[...], (((1,), (1,)), ((), ())),
                        precision=hi) + b1_ref[...], 0.0)
    combined = jnp.concatenate([h0, h1], axis=1)
    row = lax.dot_general(combined, wo_ref[...], (((1,), (1,)), ((), ())),
                          precision=hi) + bo_ref[...]
    out_ref[...] = jnp.broadcast_to(row, (B, D))


_tc_head = pl.pallas_call(
    _tc_body,
    out_shape=jax.ShapeDtypeStruct((B, D), jnp.float32),
)


def kernel(node_emb, edge_index, node_ids, W_hop0, b_hop0, W_hop1, b_hop1,
           W_out, b_out):
    masks = _sc_bfs_masks(edge_index.reshape(-1), node_ids).reshape(3, NPAD)
    return _tc_head(masks, node_emb, W_hop0, b_hop0.reshape(1, H),
                    W_hop1, b_hop1.reshape(1, H), W_out, b_out.reshape(1, D))


# speedup vs baseline: 1.0381x; 1.0381x over previous
import functools

import jax
import jax.numpy as jnp
from jax import lax
from jax.experimental import pallas as pl
from jax.experimental.pallas import tpu as pltpu
from jax.experimental.pallas import tpu_sc as plsc

N_NODES = 10000
E = 320000
D = 128
H = 128
B = 1024

NC = 1
NS = 16
NPAD = 10240
EP = E // NS
EH = EP // 2
SL = NPAD // NS
L = 16

_mesh = plsc.VectorSubcoreMesh(
    core_axis_name="c", subcore_axis_name="s", num_cores=NC, num_subcores=NS
)


@functools.partial(
    pl.kernel,
    out_type=jax.ShapeDtypeStruct((2 * NPAD,), jnp.float32),
    mesh=_mesh,
    compiler_params=pltpu.CompilerParams(needs_layout_passes=False),
    scratch_types=[
        pltpu.VMEM((EP,), jnp.int32),
        pltpu.VMEM((EP,), jnp.int32),
        pltpu.VMEM((NPAD,), jnp.int32),
        pltpu.VMEM((NPAD,), jnp.int32),
        pltpu.VMEM((NPAD,), jnp.int32),
        pltpu.VMEM((B,), jnp.int32),
        pltpu.VMEM((NS, SL), jnp.int32),
        pltpu.VMEM((SL,), jnp.int32),
        pltpu.VMEM((SL,), jnp.int32),
        pltpu.VMEM((SL,), jnp.float32),
        pltpu.VMEM((SL,), jnp.float32),
        pltpu.VMEM_SHARED((NS, NPAD), jnp.int32),
        pltpu.VMEM_SHARED((NPAD,), jnp.int32),
        pltpu.SemaphoreType.DMA,
        pltpu.SemaphoreType.DMA,
        pltpu.SemaphoreType.DMA,
    ],
)
def _sc_bfs_masks(edges_hbm, ids_hbm, out_hbm,
                  src_v, dst_v, mask0_v, f1_v, f2_v, ids_v,
                  slice_v, comb1_v, comb2_v, oa_v, ob_v,
                  stage_sh, comb_sh, sem_a, sem_b, sem_i):
    cid = lax.axis_index("c")
    tid = lax.axis_index("s")
    ebase = pl.multiple_of(tid * EP, 8)
    nbase = pl.multiple_of(tid * SL, 8)

    cp_s1 = pltpu.async_copy(edges_hbm.at[pl.ds(ebase, EH)],
                             src_v.at[pl.ds(0, EH)], sem_a)
    cp_d1 = pltpu.async_copy(edges_hbm.at[pl.ds(E + ebase, EH)],
                             dst_v.at[pl.ds(0, EH)], sem_a)
    cp_s2 = pltpu.async_copy(edges_hbm.at[pl.ds(ebase + EH, EH)],
                             src_v.at[pl.ds(EH, EH)], sem_b)
    cp_d2 = pltpu.async_copy(edges_hbm.at[pl.ds(E + ebase + EH, EH)],
                             dst_v.at[pl.ds(EH, EH)], sem_b)
    cp_i = pltpu.async_copy(ids_hbm, ids_v, sem_i)

    zeros = jnp.zeros((L,), jnp.int32)
    ones = jnp.ones((L,), jnp.int32)

    @plsc.parallel_loop(0, NPAD, step=L, unroll=8)
    def _(i):
        off = pl.multiple_of(i, 8)
        mask0_v[pl.ds(off, L)] = zeros
        f1_v[pl.ds(off, L)] = zeros
        f2_v[pl.ds(off, L)] = zeros

    cp_i.wait()

    @plsc.parallel_loop(0, B, step=L, unroll=4)
    def _(i):
        off = pl.multiple_of(i, 8)
        plsc.store_scatter(mask0_v, [ids_v[pl.ds(off, L)]], ones)

    def edge_pass(cur_ref, front_ref, lo, hi):
        @plsc.parallel_loop(lo, hi, step=L, unroll=8)
        def _(i):
            off = pl.multiple_of(i, 8)
            sv = src_v[pl.ds(off, L)]
            dv = dst_v[pl.ds(off, L)]
            fr = plsc.load_gather(cur_ref, [sv])
            plsc.store_scatter(front_ref, [dv], ones, mask=fr > 0)

    def combine(front_ref, comb_slice_ref):
        pltpu.sync_copy(front_ref, stage_sh.at[tid])
        plsc.subcore_barrier()
        pltpu.sync_copy(stage_sh.at[:, pl.ds(nbase, SL)], slice_v)

        @plsc.parallel_loop(0, SL, step=L, unroll=4)
        def _(j):
            joff = pl.multiple_of(j, 8)
            acc = slice_v[0, pl.ds(joff, L)]
            for r in range(1, NS):
                acc = jnp.maximum(acc, slice_v[r, pl.ds(joff, L)])
            comb_slice_ref[pl.ds(joff, L)] = acc

    cp_s1.wait()
    cp_d1.wait()
    edge_pass(mask0_v, f1_v, 0, EH)
    cp_s2.wait()
    cp_d2.wait()
    edge_pass(mask0_v, f1_v, EH, EP)
    combine(f1_v, comb1_v)
    pltpu.sync_copy(comb1_v, comb_sh.at[pl.ds(nbase, SL)])
    plsc.subcore_barrier()
    pltpu.sync_copy(comb_sh, f1_v)

    edge_pass(f1_v, f2_v, 0, EP)
    combine(f2_v, comb2_v)

    @plsc.parallel_loop(0, SL, step=L, unroll=4)
    def _(j):
        joff = pl.multiple_of(j, 8)
        a = comb1_v[pl.ds(joff, L)]
        u = jnp.maximum(a, comb2_v[pl.ds(joff, L)])
        oa_v[pl.ds(joff, L)] = a.astype(jnp.float32)
        ob_v[pl.ds(joff, L)] = u.astype(jnp.float32)

    pltpu.sync_copy(oa_v, out_hbm.at[pl.ds(nbase, SL)])
    ubase = pl.multiple_of(NPAD + tid * SL, 8)
    pltpu.sync_copy(ob_v, out_hbm.at[pl.ds(ubase, SL)])


def _tc_body(masks_ref, emb_ref, w0_ref, b0_ref, w1_ref, b1_ref,
             wo_ref, bo_ref, out_ref):
    hi = jax.lax.Precision.HIGHEST
    masks = masks_ref[...]
    cnt = jnp.sum(masks, axis=1)
    sums = lax.dot_general(masks[:, :N_NODES], emb_ref[...],
                           (((1,), (0,)), ((), ())), precision=hi)
    agg = jnp.where((cnt > 0)[:, None],
                    sums / jnp.maximum(cnt, 1.0)[:, None], 0.0)
    h0 = jnp.maximum(
        lax.dot_general(agg[0:1], w0_ref[...], (((1,), (1,)), ((), ())),
                        precision=hi) + b0_ref[...], 0.0)
    h1 = jnp.maximum(
        lax.dot_general(agg[1:2], w1_ref[...], (((1,), (1,)), ((), ())),
                        precision=hi) + b1_ref[...], 0.0)
    combined = jnp.concatenate([h0, h1], axis=1)
    row = lax.dot_general(combined, wo_ref[...], (((1,), (1,)), ((), ())),
                          precision=hi) + bo_ref[...]
    out_ref[...] = jnp.broadcast_to(row, (B, D))


_tc_head = pl.pallas_call(
    _tc_body,
    out_shape=jax.ShapeDtypeStruct((B, D), jnp.float32),
)


def kernel(node_emb, edge_index, node_ids, W_hop0, b_hop0, W_hop1, b_hop1,
           W_out, b_out):
    masks = _sc_bfs_masks(edge_index.reshape(-1), node_ids).reshape(2, NPAD)
    return _tc_head(masks, node_emb, W_hop0, b_hop0.reshape(1, H),
                    W_hop1, b_hop1.reshape(1, H), W_out, b_out.reshape(1, D))
